# Initial kernel scaffold; baseline (speedup 1.0000x reference)
import functools

import jax
import jax.numpy as jnp
from jax import lax
from jax.experimental import pallas as pl
from jax.experimental.pallas import tpu as pltpu
from jax.experimental.pallas import tpu_sc as plsc

B = 4096
L = 200
E = 64

NC = 2
NS = 16
NW = NC * NS

ROWS_PER_W = B // NW
CH = 8
N_CHUNKS = ROWS_PER_W // CH
HALF = L // 2
TOK_CH = CH * L


def _pool_body(texts_hbm, table_hbm, out_hbm, idx_v, rows_v, acc_v, sem):
    wid = lax.axis_index("s") * NC + lax.axis_index("c")

    def chunk_body(g, carry):
        row_base = wid * ROWS_PER_W + g * CH
        pltpu.sync_copy(texts_hbm.at[pl.ds(row_base * 2, 2 * CH)], idx_v)
        copies = []
        for j in range(2 * CH):
            copies.append(pltpu.async_copy(
                table_hbm.at[idx_v.at[j]],
                rows_v.at[pl.ds(j * HALF, HALF)],
                sem))
        for cp in copies:
            cp.wait()
        for r in range(CH):
            def tok_body(t, acc, r=r):
                base = r * L + t
                return tuple(acc[c] + rows_v[base, pl.ds(c * 16, 16)]
                             for c in range(4))
            acc = lax.fori_loop(
                0, L, tok_body,
                tuple(jnp.zeros((16,), jnp.float32) for _ in range(4)))
            for c in range(4):
                acc_v[r, pl.ds(c * 16, 16)] = acc[c] * (1.0 / L)
        pltpu.sync_copy(acc_v, out_hbm.at[pl.ds(row_base, CH)])
        return carry

    lax.fori_loop(0, N_CHUNKS, chunk_body, 0)


_pool = functools.partial(
    pl.kernel,
    out_type=jax.ShapeDtypeStruct((B, E), jnp.float32),
    mesh=plsc.VectorSubcoreMesh(core_axis_name="c", subcore_axis_name="s"),
    scratch_types=[
        pltpu.VMEM((2 * CH, HALF), jnp.int32),
        pltpu.VMEM((TOK_CH, E), jnp.float32),
        pltpu.VMEM((CH, E), jnp.float32),
        pltpu.SemaphoreType.DMA,
    ],
)(_pool_body)


def _mlp_body(p_ref, w1_ref, b1_ref, w2_ref, b2_ref, o_ref):
    h = jnp.dot(p_ref[...], w1_ref[...], preferred_element_type=jnp.float32)
    h = jnp.maximum(h + b1_ref[...], 0.0)
    o_ref[...] = (jnp.dot(h, w2_ref[...], preferred_element_type=jnp.float32)
                  + b2_ref[...])


def _mlp(pooled, W1, b1, W2, b2):
    BM = 512
    H = W1.shape[1]
    C = W2.shape[1]
    return pl.pallas_call(
        _mlp_body,
        grid=(B // BM,),
        in_specs=[
            pl.BlockSpec((BM, E), lambda i: (i, 0)),
            pl.BlockSpec((E, H), lambda i: (0, 0)),
            pl.BlockSpec((1, H), lambda i: (0, 0)),
            pl.BlockSpec((H, C), lambda i: (0, 0)),
            pl.BlockSpec((1, C), lambda i: (0, 0)),
        ],
        out_specs=pl.BlockSpec((BM, C), lambda i: (i, 0)),
        out_shape=jax.ShapeDtypeStruct((B, C), jnp.float32),
    )(pooled, W1, b1, W2, b2)


def kernel(texts, table, W1, b1, W2, b2):
    texts2 = texts.reshape(2 * B, HALF).astype(jnp.int32)
    pooled = _pool(texts2, table)
    return _mlp(pooled, W1, b1.reshape(1, -1), W2, b2.reshape(1, -1))


# speedup vs baseline: 9.8822x; 9.8822x over previous
import functools

import jax
import jax.numpy as jnp
from jax import lax
from jax.experimental import pallas as pl
from jax.experimental.pallas import tpu as pltpu
from jax.experimental.pallas import tpu_sc as plsc

B = 4096
L = 200
E = 64

NC = 2
NS = 16
NW = NC * NS

ROWS_PER_W = B // NW
CH = 8
N_CHUNKS = ROWS_PER_W // CH
HALF = L // 2
TOK_CH = CH * L


def _pool_body(texts_hbm, table_hbm, out_hbm, idx_v, rows_v, acc_v, sem):
    wid = lax.axis_index("s") * NC + lax.axis_index("c")

    def chunk_body(g, carry):
        row_base = wid * ROWS_PER_W + g * CH
        pltpu.sync_copy(texts_hbm.at[pl.ds(row_base * 2, 2 * CH)], idx_v)
        copies = []
        for j in range(2 * CH):
            copies.append(pltpu.async_copy(
                table_hbm.at[idx_v.at[j]],
                rows_v.at[pl.ds(j * HALF, HALF)],
                sem))
        for cp in copies:
            cp.wait()
        for r in range(CH):
            def tok_body(t, acc, r=r):
                base = r * L + t
                return tuple(acc[c] + rows_v[base, pl.ds(c * 16, 16)]
                             for c in range(4))
            acc = lax.fori_loop(
                0, L, tok_body,
                tuple(jnp.zeros((16,), jnp.float32) for _ in range(4)))
            for c in range(4):
                acc_v[r, pl.ds(c * 16, 16)] = acc[c] * (1.0 / L)
        pltpu.sync_copy(acc_v, out_hbm.at[pl.ds(row_base, CH)])
        return carry

    lax.fori_loop(0, N_CHUNKS, chunk_body, 0)


_pool = functools.partial(
    pl.kernel,
    out_type=jax.ShapeDtypeStruct((B, E), jnp.float32),
    mesh=plsc.VectorSubcoreMesh(core_axis_name="c", subcore_axis_name="s"),
    compiler_params=pltpu.CompilerParams(use_tc_tiling_on_sc=False),
    scratch_types=[
        pltpu.VMEM((2 * CH, HALF), jnp.int32),
        pltpu.VMEM((TOK_CH, E), jnp.float32),
        pltpu.VMEM((CH, E), jnp.float32),
        pltpu.SemaphoreType.DMA,
    ],
)(_pool_body)


def _mlp_body(p_ref, w1_ref, b1_ref, w2_ref, b2_ref, o_ref):
    h = jnp.dot(p_ref[...], w1_ref[...], preferred_element_type=jnp.float32)
    h = jnp.maximum(h + b1_ref[...], 0.0)
    o_ref[...] = (jnp.dot(h, w2_ref[...], preferred_element_type=jnp.float32)
                  + b2_ref[...])


def _mlp(pooled, W1, b1, W2, b2):
    BM = 512
    H = W1.shape[1]
    C = W2.shape[1]
    return pl.pallas_call(
        _mlp_body,
        grid=(B // BM,),
        in_specs=[
            pl.BlockSpec((BM, E), lambda i: (i, 0)),
            pl.BlockSpec((E, H), lambda i: (0, 0)),
            pl.BlockSpec((1, H), lambda i: (0, 0)),
            pl.BlockSpec((H, C), lambda i: (0, 0)),
            pl.BlockSpec((1, C), lambda i: (0, 0)),
        ],
        out_specs=pl.BlockSpec((BM, C), lambda i: (i, 0)),
        out_shape=jax.ShapeDtypeStruct((B, C), jnp.float32),
    )(pooled, W1, b1, W2, b2)


def kernel(texts, table, W1, b1, W2, b2):
    texts2 = texts.reshape(2 * B, HALF).astype(jnp.int32)
    pooled = _pool(texts2, table)
    return _mlp(pooled, W1, b1.reshape(1, -1), W2, b2.reshape(1, -1))


# speedup vs baseline: 11.4600x; 1.1597x over previous
import functools

import jax
import jax.numpy as jnp
from jax import lax
from jax.experimental import pallas as pl
from jax.experimental.pallas import tpu as pltpu
from jax.experimental.pallas import tpu_sc as plsc

B = 4096
L = 200
E = 64

NC = 2
NS = 16
NW = NC * NS

ROWS_PER_W = B // NW
CH = 8
N_CHUNKS = ROWS_PER_W // CH
HALF = L // 2
TOK_CH = CH * L


def _pool_body(texts_hbm, table_hbm, out_hbm, idx_v, rows_v, acc_v, sem):
    wid = lax.axis_index("s") * NC + lax.axis_index("c")

    def chunk_body(g, carry):
        row_base = wid * ROWS_PER_W + g * CH
        pltpu.sync_copy(texts_hbm.at[pl.ds(row_base * 2, 2 * CH)], idx_v)
        copies = []
        for j in range(2 * CH):
            copies.append(pltpu.async_copy(
                table_hbm.at[idx_v.at[j]],
                rows_v.at[pl.ds(j * HALF, HALF)],
                sem))
        for cp in copies:
            cp.wait()
        for r in range(CH):
            def tok_body(t, acc, r=r):
                base = r * L + t
                return tuple(acc[c] + rows_v[base, pl.ds(c * 16, 16)]
                             for c in range(4))
            acc = lax.fori_loop(
                0, L, tok_body,
                tuple(jnp.zeros((16,), jnp.float32) for _ in range(4)))
            for c in range(4):
                acc_v[r, pl.ds(c * 16, 16)] = acc[c] * (1.0 / L)
        pltpu.sync_copy(acc_v, out_hbm.at[pl.ds(row_base, CH)])
        return carry

    lax.fori_loop(0, N_CHUNKS, chunk_body, 0)


_pool = functools.partial(
    pl.kernel,
    out_type=jax.ShapeDtypeStruct((B, E), jnp.float32),
    mesh=plsc.VectorSubcoreMesh(core_axis_name="c", subcore_axis_name="s"),
    compiler_params=pltpu.CompilerParams(use_tc_tiling_on_sc=False),
    scratch_types=[
        pltpu.VMEM((2 * CH, HALF), jnp.int32),
        pltpu.VMEM((TOK_CH, E), jnp.float32),
        pltpu.VMEM((CH, E), jnp.float32),
        pltpu.SemaphoreType.DMA,
    ],
)(_pool_body)


def _mlp_body(pt_ref, w1t_ref, b1_ref, w2t_ref, b2_ref, ot_ref):
    ht = jnp.dot(w1t_ref[...], pt_ref[...], preferred_element_type=jnp.float32)
    ht = jnp.maximum(ht + b1_ref[...], 0.0)
    ot_ref[...] = (jnp.dot(w2t_ref[...], ht, preferred_element_type=jnp.float32)
                   + b2_ref[...])


def _mlp_t(pooled_t, W1t, b1c, W2t, b2c):
    BM = 512
    H = W1t.shape[0]
    C = W2t.shape[0]
    return pl.pallas_call(
        _mlp_body,
        grid=(B // BM,),
        in_specs=[
            pl.BlockSpec((E, BM), lambda i: (0, i)),
            pl.BlockSpec((H, E), lambda i: (0, 0)),
            pl.BlockSpec((H, 1), lambda i: (0, 0)),
            pl.BlockSpec((C, H), lambda i: (0, 0)),
            pl.BlockSpec((C, 1), lambda i: (0, 0)),
        ],
        out_specs=pl.BlockSpec((C, BM), lambda i: (0, i)),
        out_shape=jax.ShapeDtypeStruct((C, B), jnp.float32),
    )(pooled_t, W1t, b1c, W2t, b2c)


def kernel(texts, table, W1, b1, W2, b2):
    texts2 = texts.reshape(2 * B, HALF).astype(jnp.int32)
    pooled = _pool(texts2, table)
    out_t = _mlp_t(pooled.T, W1.T, b1.reshape(-1, 1), W2.T, b2.reshape(-1, 1))
    return out_t.T


# speedup vs baseline: 16.4291x; 1.4336x over previous
import functools

import jax
import jax.numpy as jnp
from jax import lax
from jax.experimental import pallas as pl
from jax.experimental.pallas import tpu as pltpu
from jax.experimental.pallas import tpu_sc as plsc

B = 4096
L = 200
E = 64

NC = 2
NS = 16
NW = NC * NS

ROWS_PER_W = B // NW
CH = 4
N_CHUNKS = ROWS_PER_W // CH
HALF = L // 2
TOK_CH = CH * L


def _pool_body(texts_hbm, table_hbm, out_hbm, idx_v, rows_v, acc_v,
               sem0, sem1):
    wid = lax.axis_index("s") * NC + lax.axis_index("c")
    sems = (sem0, sem1)

    def stage(s, g):
        row_base = wid * ROWS_PER_W + g * CH
        pltpu.sync_copy(texts_hbm.at[pl.ds(row_base * 2, 2 * CH)],
                        idx_v.at[s])
        for j in range(2 * CH):
            pltpu.async_copy(
                table_hbm.at[idx_v.at[s, j]],
                rows_v.at[s, pl.ds(j * HALF, HALF)],
                sems[s])

    def drain(s):
        pltpu.make_async_copy(
            table_hbm.at[pl.ds(0, TOK_CH)], rows_v.at[s], sems[s]).wait()

    def reduce_store(s, g):
        row_base = wid * ROWS_PER_W + g * CH
        for r in range(CH):
            def tok_body(t, acc, r=r):
                b0 = r * L + 2 * t
                new = []
                for c in range(4):
                    a = acc[c] + rows_v[s, b0, pl.ds(c * 16, 16)]
                    new.append(a + rows_v[s, b0 + 1, pl.ds(c * 16, 16)])
                return tuple(new)
            acc = lax.fori_loop(
                0, L // 2, tok_body,
                tuple(jnp.zeros((16,), jnp.float32) for _ in range(4)))
            for c in range(4):
                acc_v[r, pl.ds(c * 16, 16)] = acc[c] * (1.0 / L)
        pltpu.sync_copy(acc_v, out_hbm.at[pl.ds(row_base, CH)])

    stage(0, 0)

    def pair_body(i, carry):
        g0 = 2 * i
        stage(1, g0 + 1)
        drain(0)
        reduce_store(0, g0)

        @pl.when(g0 + 2 < N_CHUNKS)
        def _():
            stage(0, g0 + 2)

        drain(1)
        reduce_store(1, g0 + 1)
        return carry

    lax.fori_loop(0, N_CHUNKS // 2, pair_body, 0)


_pool = functools.partial(
    pl.kernel,
    out_type=jax.ShapeDtypeStruct((B, E), jnp.float32),
    mesh=plsc.VectorSubcoreMesh(core_axis_name="c", subcore_axis_name="s"),
    compiler_params=pltpu.CompilerParams(use_tc_tiling_on_sc=False),
    scratch_types=[
        pltpu.VMEM((2, 2 * CH, HALF), jnp.int32),
        pltpu.VMEM((2, TOK_CH, E), jnp.float32),
        pltpu.VMEM((CH, E), jnp.float32),
        pltpu.SemaphoreType.DMA,
        pltpu.SemaphoreType.DMA,
    ],
)(_pool_body)


def _mlp_body(pt_ref, w1t_ref, b1_ref, w2t_ref, b2_ref, ot_ref):
    ht = jnp.dot(w1t_ref[...], pt_ref[...], preferred_element_type=jnp.float32)
    ht = jnp.maximum(ht + b1_ref[...], 0.0)
    ot_ref[...] = (jnp.dot(w2t_ref[...], ht, preferred_element_type=jnp.float32)
                   + b2_ref[...])


def _mlp_t(pooled_t, W1t, b1c, W2t, b2c):
    BM = 512
    H = W1t.shape[0]
    C = W2t.shape[0]
    return pl.pallas_call(
        _mlp_body,
        grid=(B // BM,),
        in_specs=[
            pl.BlockSpec((E, BM), lambda i: (0, i)),
            pl.BlockSpec((H, E), lambda i: (0, 0)),
            pl.BlockSpec((H, 1), lambda i: (0, 0)),
            pl.BlockSpec((C, H), lambda i: (0, 0)),
            pl.BlockSpec((C, 1), lambda i: (0, 0)),
        ],
        out_specs=pl.BlockSpec((C, BM), lambda i: (0, i)),
        out_shape=jax.ShapeDtypeStruct((C, B), jnp.float32),
    )(pooled_t, W1t, b1c, W2t, b2c)


def kernel(texts, table, W1, b1, W2, b2):
    texts2 = texts.reshape(2 * B, HALF).astype(jnp.int32)
    pooled = _pool(texts2, table)
    out_t = _mlp_t(pooled.T, W1.T, b1.reshape(-1, 1), W2.T, b2.reshape(-1, 1))
    return out_t.T
